# baseline (speedup 1.0000x reference)
import functools

import jax
import jax.numpy as jnp
from jax import lax
from jax.experimental import pallas as pl
from jax.experimental.pallas import tpu as pltpu
from jax.experimental.pallas import tpu_sc as plsc

N = 10000
E = 320000
C = 128
NC, NS = 2, 16
NW = NC * NS
CHUNK = 128
G = 2
NGRP = 40
E_TILE = NGRP * G * CHUNK
E_PAD = NW * E_TILE
N_PAD = 10240
ROWS_TILE = N_PAD // NS
DEGW = 16

_sc_mesh = plsc.VectorSubcoreMesh(core_axis_name="c", subcore_axis_name="s")


@functools.partial(
    pl.kernel,
    mesh=_sc_mesh,
    out_type=(
        jax.ShapeDtypeStruct((NC, N_PAD, C), jnp.float32),
        jax.ShapeDtypeStruct((NC, N_PAD, DEGW), jnp.float32),
    ),
    scratch_types=[
        pltpu.VMEM((G, CHUNK), jnp.int32),
        pltpu.VMEM((G, CHUNK), jnp.int32),
        pltpu.VMEM((G, CHUNK), jnp.int32),
        pltpu.VMEM((G, CHUNK), jnp.int32),
        pltpu.VMEM((CHUNK, C), jnp.float32),
        pltpu.VMEM((CHUNK, C), jnp.float32),
        pltpu.VMEM((CHUNK, DEGW), jnp.float32),
        pltpu.VMEM_SHARED((N_PAD, C), jnp.float32),
        pltpu.VMEM_SHARED((N_PAD, DEGW), jnp.float32),
        pltpu.SemaphoreType.DMA,
        pltpu.SemaphoreType.DMA,
        pltpu.SemaphoreType.DMA,
        pltpu.SemaphoreType.DMA,
    ],
    compiler_params=pltpu.CompilerParams(use_tc_tiling_on_sc=False),
)
def _sc_aggregate(x_hbm, src_hbm, dst_hbm, za_hbm, zb_hbm, ones_hbm,
                  out_hbm, deg_hbm,
                  isrc0, isrc1, idst0, idst1, d0, d1, ones_v,
                  acc_sh, deg_sh, gs0, gs1, is0, is1):
    c = lax.axis_index("c")
    s = lax.axis_index("s")
    isrc = (isrc0, isrc1)
    idst = (idst0, idst1)
    dbuf = (d0, d1)
    gsem = (gs0, gs1)
    isem = (is0, is1)

    pltpu.sync_copy(za_hbm.at[pl.ds(s * ROWS_TILE, ROWS_TILE)],
                    acc_sh.at[pl.ds(s * ROWS_TILE, ROWS_TILE)])
    pltpu.sync_copy(zb_hbm.at[pl.ds(s * ROWS_TILE, ROWS_TILE)],
                    deg_sh.at[pl.ds(s * ROWS_TILE, ROWS_TILE)])
    pltpu.sync_copy(ones_hbm, ones_v)

    pltpu.sync_copy(src_hbm.at[c, s, pl.ds(0, G)], isrc0)
    pltpu.sync_copy(dst_hbm.at[c, s, pl.ds(0, G)], idst0)
    pltpu.async_copy(src_hbm.at[c, s, pl.ds(G, G)], isrc1, is1)
    pltpu.async_copy(dst_hbm.at[c, s, pl.ds(G, G)], idst1, is1)
    plsc.subcore_barrier()

    def gwait(buf, sem):
        pltpu.make_async_copy(x_hbm.at[isrc0.at[0]], buf, sem).wait()

    def iwait(p):
        pltpu.make_async_copy(src_hbm.at[0, 0, pl.ds(0, G)], isrc[p],
                              isem[p]).wait()
        pltpu.make_async_copy(dst_hbm.at[0, 0, pl.ds(0, G)], idst[p],
                              isem[p]).wait()

    pltpu.async_copy(x_hbm.at[isrc0.at[0]], d0, gs0)

    def body(gp, _):
        for gg in range(2):
            g = 2 * gp + gg
            for k in range(G):
                t = G * gg + k
                if k == G - 1:
                    iwait((gg + 1) % 2)
                gwait(dbuf[t % 2], gsem[t % 2])
                if k < G - 1:
                    nidx = isrc[gg].at[k + 1]
                else:
                    nidx = isrc[(gg + 1) % 2].at[0]
                pltpu.async_copy(x_hbm.at[nidx], dbuf[(t + 1) % 2],
                                 gsem[(t + 1) % 2])
                pltpu.sync_copy(dbuf[t % 2], acc_sh.at[idst[gg].at[k]],
                                add=True)
                pltpu.sync_copy(ones_v, deg_sh.at[idst[gg].at[k]], add=True)
            gnext = lax.rem(g + 2, NGRP)
            pltpu.async_copy(src_hbm.at[c, s, pl.ds(gnext * G, G)],
                             isrc[gg], isem[gg])
            pltpu.async_copy(dst_hbm.at[c, s, pl.ds(gnext * G, G)],
                             idst[gg], isem[gg])
        return ()

    lax.fori_loop(0, NGRP // 2, body, ())
    gwait(d0, gs0)
    iwait(1)
    plsc.subcore_barrier()
    pltpu.sync_copy(acc_sh.at[pl.ds(s * ROWS_TILE, ROWS_TILE)],
                    out_hbm.at[c].at[pl.ds(s * ROWS_TILE, ROWS_TILE)])
    pltpu.sync_copy(deg_sh.at[pl.ds(s * ROWS_TILE, ROWS_TILE)],
                    deg_hbm.at[c].at[pl.ds(s * ROWS_TILE, ROWS_TILE)])


DBLK = 1000


def _tc_dense_body(x_ref, wr_ref, b_ref, out_ref):
    dn = (((1,), (1,)), ((), ()))
    out_ref[...] = lax.dot_general(
        x_ref[...], wr_ref[...], dn,
        preferred_element_type=jnp.float32) + b_ref[...]


def _tc_dense(x, W_r, b_l):
    return pl.pallas_call(
        _tc_dense_body,
        grid=(N // DBLK,),
        in_specs=[
            pl.BlockSpec((DBLK, C), lambda i: (i, 0)),
            pl.BlockSpec((C, C), lambda i: (0, 0)),
            pl.BlockSpec((1, C), lambda i: (0, 0)),
        ],
        out_specs=pl.BlockSpec((DBLK, C), lambda i: (i, 0)),
        out_shape=jax.ShapeDtypeStruct((N, C), jnp.float32),
    )(x, W_r, b_l)


BLK = 128
CGRID = (N + BLK - 1) // BLK


def _tc_combine_body(acc_ref, deg_ref, wl_ref, dense_ref, out_ref):
    a = acc_ref[0] + acc_ref[1]
    d = deg_ref[0] + deg_ref[1]
    scale = 1.0 / jnp.maximum(d[:, 0:1], 1.0)
    agg = a * scale
    dn = (((1,), (1,)), ((), ()))
    out_ref[...] = lax.dot_general(
        agg, wl_ref[...], dn,
        preferred_element_type=jnp.float32) + dense_ref[...]


def _tc_combine(acc, deg, W_l, dense):
    return pl.pallas_call(
        _tc_combine_body,
        grid=(CGRID,),
        in_specs=[
            pl.BlockSpec((NC, BLK, C), lambda i: (0, i, 0)),
            pl.BlockSpec((NC, BLK, DEGW), lambda i: (0, i, 0)),
            pl.BlockSpec((C, C), lambda i: (0, 0)),
            pl.BlockSpec((BLK, C), lambda i: (i, 0)),
        ],
        out_specs=pl.BlockSpec((BLK, C), lambda i: (i, 0)),
        out_shape=jax.ShapeDtypeStruct((N, C), jnp.float32),
    )(acc, deg, W_l, dense)


def kernel(x, edge_index, W_l, b_l, W_r):
    pad = E_PAD - E
    src = jnp.concatenate([edge_index[0], jnp.zeros((pad,), jnp.int32)])
    pad_dst = N + jnp.arange(pad, dtype=jnp.int32) % (N_PAD - N)
    dst = jnp.concatenate([edge_index[1], pad_dst])
    src = src.reshape(NC, NS, NGRP * G, CHUNK)
    dst = dst.reshape(NC, NS, NGRP * G, CHUNK)
    za = jnp.zeros((N_PAD, C), jnp.float32)
    zb = jnp.zeros((N_PAD, DEGW), jnp.float32)
    ones = jnp.ones((CHUNK, DEGW), jnp.float32)
    dense = _tc_dense(x, W_r, b_l.reshape(1, C))
    acc, deg = _sc_aggregate(x, src, dst, za, zb, ones)
    return _tc_combine(acc, deg, W_l, dense)


# speedup vs baseline: 1.0074x; 1.0074x over previous
import functools

import jax
import jax.numpy as jnp
from jax import lax
from jax.experimental import pallas as pl
from jax.experimental.pallas import tpu as pltpu
from jax.experimental.pallas import tpu_sc as plsc

N = 10000
E = 320000
C = 128
NC, NS = 2, 16
NW = NC * NS
CHUNK = 128
G = 2
NGRP = 40
E_TILE = NGRP * G * CHUNK
E_PAD = NW * E_TILE
N_PAD = 10240
ROWS_TILE = N_PAD // NS
DEGW = 16

_sc_mesh = plsc.VectorSubcoreMesh(core_axis_name="c", subcore_axis_name="s")


@functools.partial(
    pl.kernel,
    mesh=_sc_mesh,
    out_type=(
        jax.ShapeDtypeStruct((NC, N_PAD, C), jnp.float32),
        jax.ShapeDtypeStruct((NC, N_PAD, DEGW), jnp.float32),
    ),
    scratch_types=[
        pltpu.VMEM((G, CHUNK), jnp.int32),
        pltpu.VMEM((G, CHUNK), jnp.int32),
        pltpu.VMEM((G, CHUNK), jnp.int32),
        pltpu.VMEM((G, CHUNK), jnp.int32),
        pltpu.VMEM((CHUNK, C), jnp.float32),
        pltpu.VMEM((CHUNK, C), jnp.float32),
        pltpu.VMEM((CHUNK, DEGW), jnp.float32),
        pltpu.VMEM_SHARED((N_PAD, C), jnp.float32),
        pltpu.VMEM_SHARED((N_PAD, DEGW), jnp.float32),
        pltpu.SemaphoreType.DMA,
        pltpu.SemaphoreType.DMA,
        pltpu.SemaphoreType.DMA,
        pltpu.SemaphoreType.DMA,
    ],
    compiler_params=pltpu.CompilerParams(use_tc_tiling_on_sc=False),
)
def _sc_aggregate(x_hbm, src_hbm, dst_hbm, za_hbm, zb_hbm, ones_hbm,
                  out_hbm, deg_hbm,
                  isrc0, isrc1, idst0, idst1, d0, d1, ones_v,
                  acc_sh, deg_sh, gs0, gs1, is0, is1):
    c = lax.axis_index("c")
    s = lax.axis_index("s")
    isrc = (isrc0, isrc1)
    idst = (idst0, idst1)
    dbuf = (d0, d1)
    gsem = (gs0, gs1)
    isem = (is0, is1)

    pltpu.sync_copy(za_hbm.at[pl.ds(s * ROWS_TILE, ROWS_TILE)],
                    acc_sh.at[pl.ds(s * ROWS_TILE, ROWS_TILE)])
    pltpu.sync_copy(zb_hbm.at[pl.ds(s * ROWS_TILE, ROWS_TILE)],
                    deg_sh.at[pl.ds(s * ROWS_TILE, ROWS_TILE)])
    pltpu.sync_copy(ones_hbm, ones_v)

    pltpu.sync_copy(src_hbm.at[c, s, pl.ds(0, G)], isrc0)
    pltpu.sync_copy(dst_hbm.at[c, s, pl.ds(0, G)], idst0)
    pltpu.async_copy(src_hbm.at[c, s, pl.ds(G, G)], isrc1, is1)
    pltpu.async_copy(dst_hbm.at[c, s, pl.ds(G, G)], idst1, is1)
    plsc.subcore_barrier()

    def gwait(idx, buf, sem):
        pltpu.make_async_copy(x_hbm.at[idx], buf, sem).wait()

    def iwait(p):
        pltpu.make_async_copy(src_hbm.at[c, s, pl.ds(0, G)], isrc[p],
                              isem[p]).wait()
        pltpu.make_async_copy(dst_hbm.at[c, s, pl.ds(0, G)], idst[p],
                              isem[p]).wait()

    pltpu.async_copy(x_hbm.at[isrc0.at[0]], d0, gs0)

    def body(gp, _):
        for gg in range(2):
            g = 2 * gp + gg
            for k in range(G):
                t = G * gg + k
                if k == G - 1:
                    iwait((gg + 1) % 2)
                if k < G - 1:
                    nidx = isrc[gg].at[k + 1]
                else:
                    nidx = isrc[(gg + 1) % 2].at[0]
                pltpu.async_copy(x_hbm.at[nidx], dbuf[(t + 1) % 2],
                                 gsem[(t + 1) % 2])
                gwait(isrc[gg].at[k], dbuf[t % 2], gsem[t % 2])
                pltpu.sync_copy(dbuf[t % 2], acc_sh.at[idst[gg].at[k]],
                                add=True)
                pltpu.sync_copy(ones_v, deg_sh.at[idst[gg].at[k]], add=True)
            gnext = lax.rem(g + 2, NGRP)
            pltpu.async_copy(src_hbm.at[c, s, pl.ds(gnext * G, G)],
                             isrc[gg], isem[gg])
            pltpu.async_copy(dst_hbm.at[c, s, pl.ds(gnext * G, G)],
                             idst[gg], isem[gg])
        return ()

    lax.fori_loop(0, NGRP // 2, body, ())
    gwait(isrc0.at[0], d0, gs0)
    iwait(1)
    plsc.subcore_barrier()
    pltpu.sync_copy(acc_sh.at[pl.ds(s * ROWS_TILE, ROWS_TILE)],
                    out_hbm.at[c].at[pl.ds(s * ROWS_TILE, ROWS_TILE)])
    pltpu.sync_copy(deg_sh.at[pl.ds(s * ROWS_TILE, ROWS_TILE)],
                    deg_hbm.at[c].at[pl.ds(s * ROWS_TILE, ROWS_TILE)])


DBLK = 1000


def _tc_dense_body(x_ref, wr_ref, b_ref, out_ref):
    dn = (((1,), (1,)), ((), ()))
    out_ref[...] = lax.dot_general(
        x_ref[...], wr_ref[...], dn,
        preferred_element_type=jnp.float32) + b_ref[...]


def _tc_dense(x, W_r, b_l):
    return pl.pallas_call(
        _tc_dense_body,
        grid=(N // DBLK,),
        in_specs=[
            pl.BlockSpec((DBLK, C), lambda i: (i, 0)),
            pl.BlockSpec((C, C), lambda i: (0, 0)),
            pl.BlockSpec((1, C), lambda i: (0, 0)),
        ],
        out_specs=pl.BlockSpec((DBLK, C), lambda i: (i, 0)),
        out_shape=jax.ShapeDtypeStruct((N, C), jnp.float32),
    )(x, W_r, b_l)


BLK = 128
CGRID = (N + BLK - 1) // BLK


def _tc_combine_body(acc_ref, deg_ref, wl_ref, dense_ref, out_ref):
    a = acc_ref[0] + acc_ref[1]
    d = deg_ref[0] + deg_ref[1]
    scale = 1.0 / jnp.maximum(d[:, 0:1], 1.0)
    agg = a * scale
    dn = (((1,), (1,)), ((), ()))
    out_ref[...] = lax.dot_general(
        agg, wl_ref[...], dn,
        preferred_element_type=jnp.float32) + dense_ref[...]


def _tc_combine(acc, deg, W_l, dense):
    return pl.pallas_call(
        _tc_combine_body,
        grid=(CGRID,),
        in_specs=[
            pl.BlockSpec((NC, BLK, C), lambda i: (0, i, 0)),
            pl.BlockSpec((NC, BLK, DEGW), lambda i: (0, i, 0)),
            pl.BlockSpec((C, C), lambda i: (0, 0)),
            pl.BlockSpec((BLK, C), lambda i: (i, 0)),
        ],
        out_specs=pl.BlockSpec((BLK, C), lambda i: (i, 0)),
        out_shape=jax.ShapeDtypeStruct((N, C), jnp.float32),
    )(acc, deg, W_l, dense)


def kernel(x, edge_index, W_l, b_l, W_r):
    tpad = E_TILE - E // NW
    src = jnp.concatenate(
        [edge_index[0].reshape(NW, E // NW),
         jnp.zeros((NW, tpad), jnp.int32)], axis=1)
    pad_dst = jnp.broadcast_to(
        N + jnp.arange(tpad, dtype=jnp.int32), (NW, tpad))
    dst = jnp.concatenate(
        [edge_index[1].reshape(NW, E // NW), pad_dst], axis=1)
    src = src.reshape(NC, NS, NGRP * G, CHUNK)
    dst = dst.reshape(NC, NS, NGRP * G, CHUNK)
    za = jnp.zeros((N_PAD, C), jnp.float32)
    zb = jnp.zeros((N_PAD, DEGW), jnp.float32)
    ones = jnp.ones((CHUNK, DEGW), jnp.float32)
    acc, deg = _sc_aggregate(x, src, dst, za, zb, ones)
    dense = _tc_dense(x, W_r, b_l.reshape(1, C))
    return _tc_combine(acc, deg, W_l, dense)


# speedup vs baseline: 2.0262x; 2.0114x over previous
import functools

import jax
import jax.numpy as jnp
from jax import lax
from jax.experimental import pallas as pl
from jax.experimental.pallas import tpu as pltpu
from jax.experimental.pallas import tpu_sc as plsc

N = 10000
E = 320000
C = 128
C_AUG = 144
NC, NS = 2, 16
NW = NC * NS
E_TILE = E // NW
CHUNK = 50
NCHUNK = E_TILE // CHUNK
N_PAD = 10240
ROWS_TILE = N_PAD // NS

_sc_mesh = plsc.VectorSubcoreMesh(core_axis_name="c", subcore_axis_name="s")


@functools.partial(
    pl.kernel,
    mesh=_sc_mesh,
    out_type=(
        jax.ShapeDtypeStruct((NC, N_PAD, C), jnp.float32),
        jax.ShapeDtypeStruct((NC, N_PAD, 16), jnp.float32),
    ),
    scratch_types=[
        pltpu.VMEM((NCHUNK, CHUNK), jnp.int32),
        pltpu.VMEM((NCHUNK, CHUNK), jnp.int32),
        pltpu.VMEM((CHUNK, C_AUG), jnp.float32),
        pltpu.VMEM((CHUNK, C_AUG), jnp.float32),
        pltpu.VMEM_SHARED((N_PAD, C_AUG), jnp.float32),
        pltpu.SemaphoreType.DMA,
        pltpu.SemaphoreType.DMA,
    ],
    compiler_params=pltpu.CompilerParams(use_tc_tiling_on_sc=False),
)
def _sc_aggregate(xaug_hbm, src_hbm, dst_hbm, zeros_hbm, out_hbm, deg_hbm,
                  src_v, dst_v, buf0, buf1, acc_sh, sem0, sem1):
    c = lax.axis_index("c")
    s = lax.axis_index("s")
    pltpu.sync_copy(zeros_hbm.at[pl.ds(s * ROWS_TILE, ROWS_TILE)],
                    acc_sh.at[pl.ds(s * ROWS_TILE, ROWS_TILE)])
    pltpu.sync_copy(src_hbm.at[c, s], src_v)
    pltpu.sync_copy(dst_hbm.at[c, s], dst_v)
    plsc.subcore_barrier()

    def gather(j, buf, sem):
        pltpu.async_copy(xaug_hbm.at[src_v.at[j]], buf, sem)

    def gwait(buf, sem):
        pltpu.make_async_copy(xaug_hbm.at[src_v.at[0]], buf, sem).wait()

    gather(0, buf0, sem0)

    def body(i, _):
        j0 = 2 * i
        gather(lax.rem(j0 + 1, NCHUNK), buf1, sem1)
        gwait(buf0, sem0)
        pltpu.sync_copy(buf0, acc_sh.at[dst_v.at[j0]], add=True)
        gather(lax.rem(j0 + 2, NCHUNK), buf0, sem0)
        gwait(buf1, sem1)
        pltpu.sync_copy(buf1, acc_sh.at[dst_v.at[j0 + 1]], add=True)
        return ()

    lax.fori_loop(0, NCHUNK // 2, body, ())
    gwait(buf0, sem0)
    plsc.subcore_barrier()
    pltpu.sync_copy(acc_sh.at[pl.ds(s * ROWS_TILE, ROWS_TILE), pl.ds(0, C)],
                    out_hbm.at[c].at[pl.ds(s * ROWS_TILE, ROWS_TILE)])
    pltpu.sync_copy(acc_sh.at[pl.ds(s * ROWS_TILE, ROWS_TILE), pl.ds(C, 16)],
                    deg_hbm.at[c].at[pl.ds(s * ROWS_TILE, ROWS_TILE)])


BLK = 1000


def _tc_combine_body(acc_ref, deg_ref, x_ref, wl_ref, wr_ref, b_ref, out_ref):
    a = acc_ref[0] + acc_ref[1]
    d = deg_ref[0] + deg_ref[1]
    scale = 1.0 / jnp.maximum(d[:, 0:1], 1.0)
    agg = a * scale
    dn = (((1,), (1,)), ((), ()))
    out_ref[...] = (
        lax.dot_general(agg, wl_ref[...], dn, preferred_element_type=jnp.float32)
        + lax.dot_general(x_ref[...], wr_ref[...], dn, preferred_element_type=jnp.float32)
        + b_ref[...]
    )


def _tc_combine(acc, deg, x, W_l, W_r, b_l):
    return pl.pallas_call(
        _tc_combine_body,
        grid=(N // BLK,),
        in_specs=[
            pl.BlockSpec((NC, BLK, C), lambda i: (0, i, 0)),
            pl.BlockSpec((NC, BLK, 16), lambda i: (0, i, 0)),
            pl.BlockSpec((BLK, C), lambda i: (i, 0)),
            pl.BlockSpec((C, C), lambda i: (0, 0)),
            pl.BlockSpec((C, C), lambda i: (0, 0)),
            pl.BlockSpec((1, C), lambda i: (0, 0)),
        ],
        out_specs=pl.BlockSpec((BLK, C), lambda i: (i, 0)),
        out_shape=jax.ShapeDtypeStruct((N, C), jnp.float32),
    )(acc, deg, x, W_l, W_r, b_l)


def kernel(x, edge_index, W_l, b_l, W_r):
    x_aug = jnp.concatenate(
        [x, jnp.ones((N, 1), jnp.float32), jnp.zeros((N, C_AUG - C - 1), jnp.float32)],
        axis=1)
    src = edge_index[0].reshape(NC, NS, NCHUNK, CHUNK)
    dst = edge_index[1].reshape(NC, NS, NCHUNK, CHUNK)
    zeros = jnp.zeros((N_PAD, C_AUG), jnp.float32)
    acc, deg = _sc_aggregate(x_aug, src, dst, zeros)
    return _tc_combine(acc, deg, x, W_l, W_r, b_l.reshape(1, C))


# speedup vs baseline: 2.1569x; 1.0645x over previous
import functools

import jax
import jax.numpy as jnp
from jax import lax
from jax.experimental import pallas as pl
from jax.experimental.pallas import tpu as pltpu
from jax.experimental.pallas import tpu_sc as plsc

N = 10000
E = 320000
C = 128
NC, NS = 2, 16
NW = NC * NS
E_TILE = E // NW
CHUNK = 50
NCHUNK = E_TILE // CHUNK
N_PAD = 10240
ROWS_TILE = N_PAD // NS

_sc_mesh = plsc.VectorSubcoreMesh(core_axis_name="c", subcore_axis_name="s")


@functools.partial(
    pl.kernel,
    mesh=_sc_mesh,
    out_type=(
        jax.ShapeDtypeStruct((NC, N_PAD, C), jnp.float32),
        jax.ShapeDtypeStruct((NC, N_PAD, 16), jnp.float32),
    ),
    scratch_types=[
        pltpu.VMEM((NCHUNK, CHUNK), jnp.int32),
        pltpu.VMEM((NCHUNK, CHUNK), jnp.int32),
        pltpu.VMEM((CHUNK, C), jnp.float32),
        pltpu.VMEM((CHUNK, C), jnp.float32),
        pltpu.VMEM((CHUNK, 16), jnp.float32),
        pltpu.VMEM_SHARED((N_PAD, C), jnp.float32),
        pltpu.VMEM_SHARED((N_PAD, 16), jnp.float32),
        pltpu.SemaphoreType.DMA,
        pltpu.SemaphoreType.DMA,
    ],
    compiler_params=pltpu.CompilerParams(use_tc_tiling_on_sc=False),
)
def _sc_aggregate(xaug_hbm, src_hbm, dst_hbm, za_hbm, zb_hbm, ones_hbm,
                  out_hbm, deg_hbm,
                  src_v, dst_v, buf0, buf1, ones_v, acc_sh, deg_sh,
                  sem0, sem1):
    c = lax.axis_index("c")
    s = lax.axis_index("s")
    pltpu.sync_copy(za_hbm.at[pl.ds(s * ROWS_TILE, ROWS_TILE)],
                    acc_sh.at[pl.ds(s * ROWS_TILE, ROWS_TILE)])
    pltpu.sync_copy(zb_hbm.at[pl.ds(s * ROWS_TILE, ROWS_TILE)],
                    deg_sh.at[pl.ds(s * ROWS_TILE, ROWS_TILE)])
    pltpu.sync_copy(ones_hbm, ones_v)
    pltpu.sync_copy(src_hbm.at[c, s], src_v)
    pltpu.sync_copy(dst_hbm.at[c, s], dst_v)
    plsc.subcore_barrier()

    def gather(j, buf, sem):
        pltpu.async_copy(xaug_hbm.at[src_v.at[j]], buf, sem)

    def gwait(buf, sem):
        pltpu.make_async_copy(xaug_hbm.at[src_v.at[0]], buf, sem).wait()

    gather(0, buf0, sem0)

    def body(i, _):
        j0 = 2 * i
        gather(lax.rem(j0 + 1, NCHUNK), buf1, sem1)
        gwait(buf0, sem0)
        pltpu.sync_copy(buf0, acc_sh.at[dst_v.at[j0]], add=True)
        pltpu.sync_copy(ones_v, deg_sh.at[dst_v.at[j0]], add=True)
        gather(lax.rem(j0 + 2, NCHUNK), buf0, sem0)
        gwait(buf1, sem1)
        pltpu.sync_copy(buf1, acc_sh.at[dst_v.at[j0 + 1]], add=True)
        pltpu.sync_copy(ones_v, deg_sh.at[dst_v.at[j0 + 1]], add=True)
        return ()

    lax.fori_loop(0, NCHUNK // 2, body, ())
    gwait(buf0, sem0)
    plsc.subcore_barrier()
    pltpu.sync_copy(acc_sh.at[pl.ds(s * ROWS_TILE, ROWS_TILE)],
                    out_hbm.at[c].at[pl.ds(s * ROWS_TILE, ROWS_TILE)])
    pltpu.sync_copy(deg_sh.at[pl.ds(s * ROWS_TILE, ROWS_TILE)],
                    deg_hbm.at[c].at[pl.ds(s * ROWS_TILE, ROWS_TILE)])


BLK = 1000


def _tc_combine_body(acc_ref, deg_ref, x_ref, wl_ref, wr_ref, b_ref, out_ref):
    a = acc_ref[0] + acc_ref[1]
    d = deg_ref[0] + deg_ref[1]
    scale = 1.0 / jnp.maximum(d[:, 0:1], 1.0)
    agg = a * scale
    dn = (((1,), (1,)), ((), ()))
    out_ref[...] = (
        lax.dot_general(agg, wl_ref[...], dn, preferred_element_type=jnp.float32)
        + lax.dot_general(x_ref[...], wr_ref[...], dn, preferred_element_type=jnp.float32)
        + b_ref[...]
    )


def _tc_combine(acc, deg, x, W_l, W_r, b_l):
    return pl.pallas_call(
        _tc_combine_body,
        grid=(N // BLK,),
        in_specs=[
            pl.BlockSpec((NC, BLK, C), lambda i: (0, i, 0)),
            pl.BlockSpec((NC, BLK, 16), lambda i: (0, i, 0)),
            pl.BlockSpec((BLK, C), lambda i: (i, 0)),
            pl.BlockSpec((C, C), lambda i: (0, 0)),
            pl.BlockSpec((C, C), lambda i: (0, 0)),
            pl.BlockSpec((1, C), lambda i: (0, 0)),
        ],
        out_specs=pl.BlockSpec((BLK, C), lambda i: (i, 0)),
        out_shape=jax.ShapeDtypeStruct((N, C), jnp.float32),
    )(acc, deg, x, W_l, W_r, b_l)


def kernel(x, edge_index, W_l, b_l, W_r):
    src = edge_index[0].reshape(NC, NS, NCHUNK, CHUNK)
    dst = edge_index[1].reshape(NC, NS, NCHUNK, CHUNK)
    za = jnp.zeros((N_PAD, C), jnp.float32)
    zb = jnp.zeros((N_PAD, 16), jnp.float32)
    ones = jnp.ones((CHUNK, 16), jnp.float32)
    acc, deg = _sc_aggregate(x, src, dst, za, zb, ones)
    return _tc_combine(acc, deg, x, W_l, W_r, b_l.reshape(1, C))
